# baseline (speedup 1.0000x reference)
import jax
import jax.numpy as jnp
from jax.experimental import pallas as pl
from jax.experimental.pallas import tpu as pltpu

BSZ, N_HEADS, MAX_SEQ_LEN, HEAD_DIM = 8, 16, 4096, 64
SEQLEN = 16
BH = BSZ * N_HEADS
ROWS = MAX_SEQ_LEN * HEAD_DIM // 128
UROWS = SEQLEN * HEAD_DIM // 128


def _body(pos_ref, prev_ref, u_ref, o_ref, stage, t0, t8, sem, s0, s8):
    del prev_ref
    p = pos_ref[0]
    r0 = p // 2

    @pl.when(p % 2 == 0)
    def _even():
        cp = pltpu.make_async_copy(u_ref, o_ref.at[:, pl.ds(r0, UROWS), :], sem)
        cp.start()
        cp.wait()

    @pl.when(p % 2 == 1)
    def _odd():
        g0 = pltpu.make_async_copy(o_ref.at[:, pl.ds(r0, 1), :], t0, s0)
        g8 = pltpu.make_async_copy(o_ref.at[:, pl.ds(r0 + UROWS, 1), :], t8, s8)
        g0.start()
        g8.start()
        g0.wait()
        g8.wait()
        u = u_ref[...]
        row0 = jnp.concatenate([t0[...][:, 0, 0:64], u[:, 0, 0:64]], axis=-1)
        mid = jnp.concatenate([u[:, 0:7, 64:128], u[:, 1:8, 0:64]], axis=-1)
        row8 = jnp.concatenate([u[:, 7, 64:128], t8[...][:, 0, 64:128]], axis=-1)
        stage[...] = jnp.concatenate(
            [row0[:, None, :], mid, row8[:, None, :]], axis=1
        )
        cp = pltpu.make_async_copy(
            stage, o_ref.at[:, pl.ds(r0, UROWS + 1), :], sem
        )
        cp.start()
        cp.wait()


def kernel(cache, update, pos):
    c3 = cache.reshape(BH, ROWS, 128)
    u3 = update.reshape(BH, UROWS, 128)
    out = pl.pallas_call(
        _body,
        grid_spec=pltpu.PrefetchScalarGridSpec(
            num_scalar_prefetch=1,
            grid=(1,),
            in_specs=[
                pl.BlockSpec(memory_space=pl.ANY),
                pl.BlockSpec((BH, UROWS, 128), lambda i, p: (0, 0, 0)),
            ],
            out_specs=pl.BlockSpec(memory_space=pl.ANY),
            scratch_shapes=[
                pltpu.VMEM((BH, UROWS + 1, 128), jnp.float32),
                pltpu.VMEM((BH, 1, 128), jnp.float32),
                pltpu.VMEM((BH, 1, 128), jnp.float32),
                pltpu.SemaphoreType.DMA,
                pltpu.SemaphoreType.DMA,
                pltpu.SemaphoreType.DMA,
            ],
        ),
        out_shape=jax.ShapeDtypeStruct((BH, ROWS, 128), jnp.float32),
        input_output_aliases={1: 0},
    )(pos, c3, u3)
    return out.reshape(BSZ, N_HEADS, MAX_SEQ_LEN, HEAD_DIM)


# speedup vs baseline: 2.9314x; 2.9314x over previous
import jax
import jax.numpy as jnp
from jax.experimental import pallas as pl
from jax.experimental.pallas import tpu as pltpu

BSZ, N_HEADS, MAX_SEQ_LEN, HEAD_DIM = 8, 16, 4096, 64
SEQLEN = 16
BH = BSZ * N_HEADS


def _upd_body(pos_ref, prev_ref, u_ref, o_ref, sem):
    del prev_ref
    p = pos_ref[0]
    cp = pltpu.make_async_copy(u_ref, o_ref.at[:, pl.ds(p, SEQLEN), :], sem)
    cp.start()
    cp.wait()


def kernel(cache, update, pos):
    c3 = cache.reshape(BH, MAX_SEQ_LEN, HEAD_DIM)
    u3 = update.reshape(BH, SEQLEN, HEAD_DIM)
    out = pl.pallas_call(
        _upd_body,
        grid_spec=pltpu.PrefetchScalarGridSpec(
            num_scalar_prefetch=1,
            grid=(1,),
            in_specs=[
                pl.BlockSpec(memory_space=pl.ANY),
                pl.BlockSpec((BH, SEQLEN, HEAD_DIM), lambda i, p: (0, 0, 0)),
            ],
            out_specs=pl.BlockSpec(memory_space=pl.ANY),
            scratch_shapes=[pltpu.SemaphoreType.DMA],
        ),
        out_shape=jax.ShapeDtypeStruct((BH, MAX_SEQ_LEN, HEAD_DIM), jnp.float32),
        input_output_aliases={1: 0},
    )(pos, c3, u3)
    return out.reshape(BSZ, N_HEADS, MAX_SEQ_LEN, HEAD_DIM)
